# baseline (speedup 1.0000x reference)
import functools

import jax
import jax.numpy as jnp
from jax import lax
from jax.experimental import pallas as pl
from jax.experimental.pallas import tpu as pltpu
from jax.experimental.pallas import tpu_sc as plsc

_NUM_WORKERS = 32
_CHUNK = 512


@functools.partial(jax.jit, static_argnums=(2, 3))
def _gather_rows(idx, table, n, d):
    n_per_w = n // _NUM_WORKERS
    n_chunks = n_per_w // _CHUNK
    mesh = plsc.VectorSubcoreMesh(core_axis_name="c", subcore_axis_name="s")

    @functools.partial(
        pl.kernel,
        mesh=mesh,
        out_type=jax.ShapeDtypeStruct((n, d), jnp.float32),
        scratch_types=[
            pltpu.VMEM((_CHUNK,), jnp.int32),
            pltpu.VMEM((_CHUNK, d), jnp.float32),
            pltpu.SemaphoreType.DMA,
        ],
        compiler_params=pltpu.CompilerParams(use_tc_tiling_on_sc=False),
    )
    def k(idx_hbm, table_hbm, out_hbm, idx_v, rows_v, sem):
        wid = lax.axis_index("s") * 2 + lax.axis_index("c")
        base = wid * n_per_w

        def body(i, _):
            off = base + i * _CHUNK
            pltpu.sync_copy(idx_hbm.at[pl.ds(off, _CHUNK)], idx_v)
            pltpu.async_copy(table_hbm.at[idx_v], rows_v, sem).wait()
            pltpu.sync_copy(rows_v, out_hbm.at[pl.ds(off, _CHUNK)])
            return 0

        lax.fori_loop(0, n_chunks, body, 0)

    return k(idx, table)


def kernel(sentence, table):
    b, s = sentence.shape
    v, d = table.shape
    n = b * s
    idx = sentence.reshape(n).astype(jnp.int32)
    out = _gather_rows(idx, table, n, d)
    return out.reshape(b, s, d)


# speedup vs baseline: 1.0376x; 1.0376x over previous
import functools

import jax
import jax.numpy as jnp
from jax import lax
from jax.experimental import pallas as pl
from jax.experimental.pallas import tpu as pltpu
from jax.experimental.pallas import tpu_sc as plsc

_NUM_WORKERS = 32
_CHUNK = 512
_NBUF = 2


@functools.partial(jax.jit, static_argnums=(2, 3))
def _gather_rows(idx, table, n, d):
    n_per_w = n // _NUM_WORKERS
    n_chunks = n_per_w // _CHUNK
    n_outer = n_chunks // _NBUF
    mesh = plsc.VectorSubcoreMesh(core_axis_name="c", subcore_axis_name="s")

    @functools.partial(
        pl.kernel,
        mesh=mesh,
        out_type=jax.ShapeDtypeStruct((n, d), jnp.float32),
        scratch_types=(
            [pltpu.VMEM((_CHUNK,), jnp.int32) for _ in range(_NBUF)]
            + [pltpu.VMEM((_CHUNK, d), jnp.float32) for _ in range(_NBUF)]
            + [pltpu.SemaphoreType.DMA for _ in range(3 * _NBUF)]
        ),
        compiler_params=pltpu.CompilerParams(use_tc_tiling_on_sc=False),
    )
    def k(idx_hbm, table_hbm, out_hbm, *scratch):
        idx_v = scratch[:_NBUF]
        rows_v = scratch[_NBUF:2 * _NBUF]
        sem_i = scratch[2 * _NBUF:3 * _NBUF]
        sem_g = scratch[3 * _NBUF:4 * _NBUF]
        sem_s = scratch[4 * _NBUF:5 * _NBUF]

        wid = lax.axis_index("s") * 2 + lax.axis_index("c")
        base = wid * n_per_w

        def idx_load(chunk, b):
            pltpu.async_copy(
                idx_hbm.at[pl.ds(base + chunk * _CHUNK, _CHUNK)],
                idx_v[b], sem_i[b])

        for b in range(_NBUF):
            idx_load(b, b)

        def outer(g, _):
            for b in range(_NBUF):
                chunk = g * _NBUF + b

                @pl.when(g > 0)
                def _wait_store():
                    pltpu.make_async_copy(
                        rows_v[b],
                        out_hbm.at[pl.ds(base, _CHUNK)],
                        sem_s[b]).wait()

                pltpu.make_async_copy(
                    idx_hbm.at[pl.ds(base, _CHUNK)],
                    idx_v[b], sem_i[b]).wait()
                pltpu.async_copy(table_hbm.at[idx_v[b]], rows_v[b], sem_g[b])

            for b in range(_NBUF):
                chunk = g * _NBUF + b
                pltpu.make_async_copy(
                    table_hbm.at[idx_v[b]], rows_v[b], sem_g[b]).wait()
                pltpu.async_copy(
                    rows_v[b],
                    out_hbm.at[pl.ds(base + chunk * _CHUNK, _CHUNK)],
                    sem_s[b])

                @pl.when(g < n_outer - 1)
                def _next_idx():
                    idx_load((g + 1) * _NBUF + b, b)

            return 0

        lax.fori_loop(0, n_outer, outer, 0)

        for b in range(_NBUF):
            pltpu.make_async_copy(
                rows_v[b],
                out_hbm.at[pl.ds(base, _CHUNK)],
                sem_s[b]).wait()

    return k(idx, table)


def kernel(sentence, table):
    b, s = sentence.shape
    v, d = table.shape
    n = b * s
    idx = sentence.reshape(n).astype(jnp.int32)
    out = _gather_rows(idx, table, n, d)
    return out.reshape(b, s, d)
